# baseline (speedup 1.0000x reference)
import dataclasses

import jax
import jax.numpy as jnp
from jax import lax
from jax.experimental import pallas as pl
from jax.experimental.pallas import tpu as pltpu
from jax.experimental.pallas import tpu_sc as plsc

B = 1024
IN_F = 4096
OUT_F = 4096
K = 16
LANES = 16

NP2 = B // 2
NW = 32
NT = OUT_F // NW
SBN = 4
NSB = NT // SBN
WCHUNK = NP2 // LANES
OBLK = 16
OPITCH = OBLK + 1


def _body(pairs_hbm, wexp_hbm, bexp_hbm, maskf_hbm, out_hbm,
          rows0, rows1, w_v, bias_v, m_v, out_v, sem0, sem1):
    c = lax.axis_index("c")
    s = lax.axis_index("s")
    wid = s * 2 + c
    n0 = wid * NT

    pltpu.sync_copy(maskf_hbm.at[pl.ds(n0 * K, NT * K)], m_v)
    pltpu.sync_copy(wexp_hbm.at[pl.ds(n0, NT)], w_v)
    pltpu.sync_copy(bexp_hbm.at[pl.ds(n0, NT)], bias_v)

    def gather(sb, buf, sem):
        idx = m_v.at[pl.ds(sb * (SBN * K), SBN * K)]
        pltpu.async_copy(pairs_hbm.at[idx], buf, sem)

    def wait_rows(sb, buf, sem):
        idx = m_v.at[pl.ds(sb * (SBN * K), SBN * K)]
        pltpu.make_async_copy(pairs_hbm.at[idx], buf, sem).wait()

    gather(0, rows0, sem0)
    gather(1, rows1, sem1)

    iota2 = lax.iota(jnp.int32, LANES) * 2

    def compute(sb, buf):
        for nj in range(SBN):
            n_loc = sb * SBN + nj
            nst = n_loc % OBLK
            nstv = jnp.full((LANES,), nst, jnp.int32)
            wvecs = [w_v[n_loc, k] for k in range(K)]
            bvec = bias_v[n_loc]

            def wc_body(wc, carry):
                acc_e = bvec
                acc_o = bvec
                for k in range(K):
                    word = buf[nj * K + k, pl.ds(wc * LANES, LANES)]
                    both = plsc.bitcast(word, jnp.bfloat16)
                    xlo, xhi = plsc.unpack(
                        both, format=plsc.PackFormat.INTERLEAVED)
                    acc_e = acc_e + xlo * wvecs[k]
                    acc_o = acc_o + xhi * wvecs[k]
                bv = iota2 + wc * (2 * LANES)
                plsc.store_scatter(out_v, [bv, nstv], acc_e)
                plsc.store_scatter(out_v, [bv + 1, nstv], acc_o)
                return carry

            lax.fori_loop(0, WCHUNK, wc_body, 0)

    def sb_pair(sbp, carry):
        sb0 = sbp * 2
        wait_rows(sb0, rows0, sem0)
        compute(sb0, rows0)

        @pl.when(sb0 + 2 < NSB)
        def _pf0():
            gather(sb0 + 2, rows0, sem0)

        wait_rows(sb0 + 1, rows1, sem1)
        compute(sb0 + 1, rows1)

        @pl.when(sb0 + 3 < NSB)
        def _pf1():
            gather(sb0 + 3, rows1, sem1)

        @pl.when(sbp % 2 == 1)
        def _flush():
            ob = sbp // 2
            pltpu.sync_copy(
                out_v.at[:, pl.ds(0, OBLK)],
                out_hbm.at[:, pl.ds(n0 + ob * OBLK, OBLK)])
        return carry

    lax.fori_loop(0, NSB // 2, sb_pair, 0)


@jax.jit
def kernel(input, condensed_weight, bias, input_mask):
    pairsT = jax.lax.bitcast_convert_type(
        input.astype(jnp.bfloat16).T.reshape(IN_F, B // 2, 2), jnp.int32)
    wexp = jnp.broadcast_to(condensed_weight[:, :, None], (OUT_F, K, LANES))
    bexp = jnp.broadcast_to(bias[:, None], (OUT_F, LANES))
    maskf = input_mask.astype(jnp.int32).reshape(-1)
    mesh = plsc.VectorSubcoreMesh(core_axis_name="c", subcore_axis_name="s")
    cp = pltpu.CompilerParams()
    if "needs_layout_passes" in pltpu.CompilerParams.__dataclass_fields__:
        cp = dataclasses.replace(cp, needs_layout_passes=False)
    cp = dataclasses.replace(cp, use_tc_tiling_on_sc=False)
    f = pl.kernel(
        _body,
        out_type=jax.ShapeDtypeStruct((B, OUT_F), jnp.float32),
        mesh=mesh,
        scratch_types=[
            pltpu.VMEM((SBN * K, NP2), jnp.int32),
            pltpu.VMEM((SBN * K, NP2), jnp.int32),
            pltpu.VMEM((NT, K, LANES), jnp.float32),
            pltpu.VMEM((NT, LANES), jnp.float32),
            pltpu.VMEM((NT * K,), jnp.int32),
            pltpu.VMEM((B, OPITCH), jnp.float32),
            pltpu.SemaphoreType.DMA,
            pltpu.SemaphoreType.DMA,
        ],
        compiler_params=cp,
    )
    return f(pairsT, wexp, bexp, maskf)
